# baseline (speedup 1.0000x reference)
import jax
import jax.numpy as jnp
from jax import lax
from jax.experimental import pallas as pl
from jax.experimental.pallas import tpu as pltpu
from jax.experimental.pallas import tpu_sc as plsc

_NC, _NS = 2, 16
_NW = _NC * _NS
_D = 1024
_LANES = 16
_CHUNK = 32
_NBUF = 3


def _sc_body(x_hbm, idx_hbm, emb_hbm, out_hbm,
             idx_v, emb_v, scale_v, xb0, xb1, xb2,
             si0, si1, si2, so0, so1, so2):
    rows = x_hbm.shape[0]
    seq = idx_hbm.shape[0]
    rpw = rows // _NW
    nchunk = rpw // _CHUNK
    bufs = (xb0, xb1, xb2)
    in_sems = (si0, si1, si2)
    out_sems = (so0, so1, so2)

    wid = lax.axis_index("s") * _NC + lax.axis_index("c")
    base = wid * rpw

    pltpu.sync_copy(idx_hbm.at[pl.ds(lax.rem(base, seq), rpw)], idx_v)
    pltpu.sync_copy(emb_hbm, emb_v)

    for k in range(rpw // _LANES):
        sl = pl.ds(_LANES * k, _LANES)
        scale_v[sl] = (idx_v[sl] - 1).astype(jnp.float32)

    def in_desc(i):
        b = i % _NBUF
        src = x_hbm.at[pl.ds(base + _CHUNK * i, _CHUNK)]
        return pltpu.make_async_copy(src, bufs[b], in_sems[b])

    def out_desc(i):
        b = i % _NBUF
        dst = out_hbm.at[pl.ds(base + _CHUNK * i, _CHUNK)]
        return pltpu.make_async_copy(bufs[b], dst, out_sems[b])

    def compute(i):
        buf = bufs[i % _NBUF]

        def row_body(r, carry):
            lr = _CHUNK * i + r
            s_v = plsc.load_gather(scale_v, [jnp.full((_LANES,), lr, jnp.int32)])
            for j in range(_D // _LANES):
                sl = pl.ds(_LANES * j, _LANES)
                buf[r, sl] = buf[r, sl] + s_v * emb_v[sl]
            return carry

        lax.fori_loop(0, _CHUNK, row_body, 0)

    for i in range(min(_NBUF, nchunk)):
        in_desc(i).start()
    for i in range(nchunk):
        wait_in = in_desc(i)
        wait_in.wait()
        compute(i)
        out_desc(i).start()
        nxt = i + _NBUF
        if nxt < nchunk:
            out_desc(i).wait()
            in_desc(nxt).start()
    for i in range(max(0, nchunk - _NBUF), nchunk):
        out_desc(i).wait()


def kernel(x, indices, emb_weight):
    B, S, D = x.shape
    xf = x.reshape(B * S, D)
    emb1 = emb_weight.reshape(D)
    mesh = plsc.VectorSubcoreMesh(core_axis_name="c", subcore_axis_name="s")
    out = pl.kernel(
        _sc_body,
        out_type=jax.ShapeDtypeStruct((B * S, D), x.dtype),
        mesh=mesh,
        compiler_params=pltpu.CompilerParams(needs_layout_passes=False),
        scratch_types=[
            pltpu.VMEM((B * S // _NW,), jnp.int32),
            pltpu.VMEM((D,), jnp.float32),
            pltpu.VMEM((B * S // _NW,), jnp.float32),
            pltpu.VMEM((_CHUNK, D), jnp.float32),
            pltpu.VMEM((_CHUNK, D), jnp.float32),
            pltpu.VMEM((_CHUNK, D), jnp.float32),
            pltpu.SemaphoreType.DMA,
            pltpu.SemaphoreType.DMA,
            pltpu.SemaphoreType.DMA,
            pltpu.SemaphoreType.DMA,
            pltpu.SemaphoreType.DMA,
            pltpu.SemaphoreType.DMA,
        ],
    )(xf, indices, emb1)
    return out.reshape(B, S, D)


# speedup vs baseline: 3.3254x; 3.3254x over previous
import jax
import jax.numpy as jnp
from jax import lax
from jax.experimental import pallas as pl
from jax.experimental.pallas import tpu as pltpu
from jax.experimental.pallas import tpu_sc as plsc

_NC, _NS = 2, 16
_NW = _NC * _NS
_LANES = 16
_SEQ_BLOCK = 2048


def _scale_body(idx_hbm, scale_hbm, idx_v, scale_v):
    seq = idx_hbm.shape[0]
    n = seq // _NW
    wid = lax.axis_index("s") * _NC + lax.axis_index("c")
    base = wid * n
    pltpu.sync_copy(idx_hbm.at[pl.ds(base, n)], idx_v)
    for k in range(n // _LANES):
        sl = pl.ds(_LANES * k, _LANES)
        scale_v[sl] = (idx_v[sl] - 1).astype(jnp.float32)
    pltpu.sync_copy(scale_v, scale_hbm.at[pl.ds(base, n)])


def _sc_scales(indices):
    (seq,) = indices.shape
    mesh = plsc.VectorSubcoreMesh(core_axis_name="c", subcore_axis_name="s")
    return pl.kernel(
        _scale_body,
        out_type=jax.ShapeDtypeStruct((seq,), jnp.float32),
        mesh=mesh,
        compiler_params=pltpu.CompilerParams(needs_layout_passes=False),
        scratch_types=[
            pltpu.VMEM((seq // _NW,), jnp.int32),
            pltpu.VMEM((seq // _NW,), jnp.float32),
        ],
    )(indices)


def _tc_body(scale_ref, emb_ref, x_ref, o_ref):
    scale = scale_ref[0, 0, :]
    o_ref[...] = x_ref[...] + (scale[:, None] * emb_ref[0][None, :])[None]


def kernel(x, indices, emb_weight):
    B, S, D = x.shape
    ns = S // _SEQ_BLOCK
    scale3 = _sc_scales(indices).reshape(ns, 1, _SEQ_BLOCK)
    return pl.pallas_call(
        _tc_body,
        grid=(B, ns),
        in_specs=[
            pl.BlockSpec((1, 1, _SEQ_BLOCK), lambda b, s: (s, 0, 0)),
            pl.BlockSpec((1, D), lambda b, s: (0, 0)),
            pl.BlockSpec((1, _SEQ_BLOCK, D), lambda b, s: (b, s, 0)),
        ],
        out_specs=pl.BlockSpec((1, _SEQ_BLOCK, D), lambda b, s: (b, s, 0)),
        out_shape=jax.ShapeDtypeStruct((B, S, D), x.dtype),
        compiler_params=pltpu.CompilerParams(
            dimension_semantics=("parallel", "parallel"),
        ),
    )(scale3, emb_weight, x)


# speedup vs baseline: 6.5630x; 1.9736x over previous
import jax
import jax.numpy as jnp
from jax.experimental import pallas as pl
from jax.experimental.pallas import tpu as pltpu

_ROW_BLOCK = 3840


def _body(idx_ref, emb_ref, x_ref, o_ref):
    scale = (idx_ref[0, 0, :] - 1).astype(jnp.float32)
    o_ref[...] = x_ref[...] + scale[:, None] * emb_ref[0][None, :]


def kernel(x, indices, emb_weight):
    B, S, D = x.shape
    rows = B * S
    xf = x.reshape(rows, D)
    nb = pl.cdiv(rows, _ROW_BLOCK)
    idx_flat = jnp.tile(indices, B)
    idx_pad = jnp.pad(idx_flat, (0, nb * _ROW_BLOCK - rows))
    idx3 = idx_pad.reshape(nb, 1, _ROW_BLOCK)
    out = pl.pallas_call(
        _body,
        grid=(nb,),
        in_specs=[
            pl.BlockSpec((1, 1, _ROW_BLOCK), lambda i: (i, 0, 0)),
            pl.BlockSpec((1, D), lambda i: (0, 0)),
            pl.BlockSpec((_ROW_BLOCK, D), lambda i: (i, 0)),
        ],
        out_specs=pl.BlockSpec((_ROW_BLOCK, D), lambda i: (i, 0)),
        out_shape=jax.ShapeDtypeStruct((rows, D), x.dtype),
        compiler_params=pltpu.CompilerParams(
            dimension_semantics=("parallel",),
            vmem_limit_bytes=63 * 1024 * 1024,
        ),
    )(idx3, emb_weight, xf)
    return out.reshape(B, S, D)
